# baseline (speedup 1.0000x reference)
import functools

import jax
import jax.numpy as jnp
from jax import lax
from jax.experimental import pallas as pl
from jax.experimental.pallas import tpu as pltpu
from jax.experimental.pallas import tpu_sc as plsc

_C0 = 20000
_C1 = 60000
_C2 = 100000
_D0, _D1, _D2 = 1024, 256, 64
_D2P = 128
_OUT = 1024

_NC, _NS, _L = 2, 16, 16
_NW = _NC * _NS
_B = 8192
_BPW = _B // _NW
_CH0 = 32
_CH12 = 64
_NC0 = _BPW // _CH0
_NC12 = _BPW // _CH12


def _band_ring(base, chunk, nchunk, idx_ref, table, x_hbm, bufs, gsem, wsem):
    g = [None] * nchunk
    w = [None] * nchunk

    def fire_gather(c):
        return pltpu.async_copy(table.at[idx_ref.at[c]], bufs[c % 2], gsem)

    def fire_write(c):
        return pltpu.async_copy(bufs[c % 2],
                                x_hbm.at[pl.ds(base + c * chunk, chunk)], wsem)

    g[0] = fire_gather(0)
    for c in range(nchunk):
        if c + 1 < nchunk:
            if c >= 1:
                w[c - 1].wait()
            g[c + 1] = fire_gather(c + 1)
        g[c].wait()
        w[c] = fire_write(c)
    w[nchunk - 2].wait()
    w[nchunk - 1].wait()


def _sc_gather_body(ids_hbm, emb0, emb1, emb2, x0_hbm, x1_hbm, x2_hbm,
                    ids_v, i0_v, i1_v, i2_v,
                    r0a, r0b, r1a, r1b, r2a, r2b, gsem, wsem):
    wid = lax.axis_index("s") * _NC + lax.axis_index("c")
    base = wid * _BPW
    pltpu.sync_copy(ids_hbm.at[pl.ds(base, _BPW)], ids_v)
    for g in range(_BPW // _L):
        s = g * _L
        t = ids_v[pl.ds(s, _L)]
        i0_v[s // _CH0, pl.ds(s % _CH0, _L)] = jnp.clip(t, 0, _C0 - 1)
        i1_v[s // _CH12, pl.ds(s % _CH12, _L)] = jnp.clip(t - _C0, 0, (_C1 - _C0) - 1)
        i2_v[s // _CH12, pl.ds(s % _CH12, _L)] = jnp.clip(t - _C1, 0, (_C2 - _C1) - 1)

    _band_ring(base, _CH0, _NC0, i0_v, emb0, x0_hbm, (r0a, r0b), gsem, wsem)
    _band_ring(base, _CH12, _NC12, i1_v, emb1, x1_hbm, (r1a, r1b), gsem, wsem)
    _band_ring(base, _CH12, _NC12, i2_v, emb2, x2_hbm, (r2a, r2b), gsem, wsem)


_sc_gather = pl.kernel(
    _sc_gather_body,
    out_type=(
        jax.ShapeDtypeStruct((_B, _D0), jnp.float32),
        jax.ShapeDtypeStruct((_B, _D1), jnp.float32),
        jax.ShapeDtypeStruct((_B, _D2P), jnp.float32),
    ),
    mesh=plsc.VectorSubcoreMesh(core_axis_name="c", subcore_axis_name="s"),
    scratch_types=[
        pltpu.VMEM((_BPW,), jnp.int32),
        pltpu.VMEM((_NC0, _CH0), jnp.int32),
        pltpu.VMEM((_NC12, _CH12), jnp.int32),
        pltpu.VMEM((_NC12, _CH12), jnp.int32),
        pltpu.VMEM((_CH0, _D0), jnp.float32),
        pltpu.VMEM((_CH0, _D0), jnp.float32),
        pltpu.VMEM((_CH12, _D1), jnp.float32),
        pltpu.VMEM((_CH12, _D1), jnp.float32),
        pltpu.VMEM((_CH12, _D2P), jnp.float32),
        pltpu.VMEM((_CH12, _D2P), jnp.float32),
        pltpu.SemaphoreType.DMA,
        pltpu.SemaphoreType.DMA,
    ],
)

_BT = 512


def _tc_body(ids_ref, x0_ref, x1_ref, x2_ref, w0_ref, w1_ref, w2_ref, out_ref):
    t = ids_ref[...]
    m0 = (t < _C0).astype(jnp.bfloat16)
    m1 = jnp.logical_and(t >= _C0, t < _C1).astype(jnp.bfloat16)
    m2 = (t >= _C1).astype(jnp.bfloat16)
    a0 = x0_ref[...].astype(jnp.bfloat16) * m0
    a1 = x1_ref[...].astype(jnp.bfloat16) * m1
    a2 = x2_ref[...].astype(jnp.bfloat16) * m2
    acc = jnp.dot(a0, w0_ref[...], preferred_element_type=jnp.float32)
    acc += jnp.dot(a1, w1_ref[...], preferred_element_type=jnp.float32)
    acc += jnp.dot(a2, w2_ref[...], preferred_element_type=jnp.float32)
    out_ref[...] = acc


@functools.partial(jax.jit, static_argnames=())
def _run(ids, emb0, w0t, emb1, w1t, emb2, w2t):
    x0, x1, x2 = _sc_gather(ids, emb0, emb1, emb2)
    ids2d = ids.reshape(_B, 1)
    grid = _B // _BT
    out = pl.pallas_call(
        _tc_body,
        grid=(grid,),
        in_specs=[
            pl.BlockSpec((_BT, 1), lambda i: (i, 0)),
            pl.BlockSpec((_BT, _D0), lambda i: (i, 0)),
            pl.BlockSpec((_BT, _D1), lambda i: (i, 0)),
            pl.BlockSpec((_BT, _D2P), lambda i: (i, 0)),
            pl.BlockSpec((_D0, _OUT), lambda i: (0, 0)),
            pl.BlockSpec((_D1, _OUT), lambda i: (0, 0)),
            pl.BlockSpec((_D2P, _OUT), lambda i: (0, 0)),
        ],
        out_specs=pl.BlockSpec((_BT, _OUT), lambda i: (i, 0)),
        out_shape=jax.ShapeDtypeStruct((_B, _OUT), jnp.float32),
    )(ids2d, x0, x1, x2, w0t, w1t, w2t)
    return out


def kernel(input, emb0, W0, emb1, W1, emb2, W2):
    ids = input.reshape(-1).astype(jnp.int32)
    emb2p = jnp.pad(emb2, ((0, 0), (0, _D2P - _D2)))
    w2tp = jnp.pad(W2.T, ((0, _D2P - _D2), (0, 0)))
    out = _run(ids, emb0, W0.T.astype(jnp.bfloat16), emb1,
               W1.T.astype(jnp.bfloat16), emb2p, w2tp.astype(jnp.bfloat16))
    return out.reshape(input.shape + (_OUT,))
